# baseline (speedup 1.0000x reference)
import functools

import jax
import jax.numpy as jnp
from jax import lax
from jax.experimental import pallas as pl
from jax.experimental.pallas import tpu as pltpu
from jax.experimental.pallas import tpu_sc as plsc


def _argmin_body(x_ref, c_ref, assign_ref, counts_out_ref, c2_scr, *,
                 num_blocks, num_clusters, bn):
    i = pl.program_id(0)

    @pl.when(i == 0)
    def _init():
        cc = c_ref[...]
        c2_scr[...] = jnp.broadcast_to(
            jnp.sum(cc * cc, axis=1, keepdims=True), c2_scr.shape)
        counts_out_ref[...] = jnp.zeros_like(counts_out_ref)

    x = x_ref[...]
    scores = lax.dot_general(
        c_ref[...], x, (((1,), (1,)), ((), ())),
        preferred_element_type=jnp.float32)
    val = scores - 0.5 * c2_scr[:, 0:1]
    mx = jnp.max(val, axis=0, keepdims=True)
    iota_c = lax.broadcasted_iota(jnp.int32, (num_clusters, bn), 0)
    assign = jnp.min(jnp.where(val == mx, iota_c, num_clusters), axis=0)
    assign_ref[...] = assign
    onehot = (iota_c == assign[None, :]).astype(jnp.float32)
    cnt = jnp.sum(onehot, axis=1, keepdims=True)
    counts_out_ref[...] += jnp.broadcast_to(cnt, counts_out_ref.shape)


def _tc_argmin(x, centers):
    n, dim = x.shape
    num_clusters = centers.shape[0]
    bn = 512
    num_blocks = n // bn
    return pl.pallas_call(
        functools.partial(_argmin_body, num_blocks=num_blocks,
                          num_clusters=num_clusters, bn=bn),
        grid=(num_blocks,),
        in_specs=[
            pl.BlockSpec((bn, dim), lambda i: (i, 0)),
            pl.BlockSpec((num_clusters, dim), lambda i: (0, 0)),
        ],
        out_specs=[
            pl.BlockSpec((bn,), lambda i: (i,)),
            pl.BlockSpec((num_clusters, 8), lambda i: (0, 0)),
        ],
        out_shape=[
            jax.ShapeDtypeStruct((n,), jnp.int32),
            jax.ShapeDtypeStruct((num_clusters, 8), jnp.float32),
        ],
        scratch_shapes=[pltpu.VMEM((num_clusters, 8), jnp.float32)],
        compiler_params=pltpu.CompilerParams(
            dimension_semantics=("arbitrary",)),
    )(x, centers)


_CHUNK = 256
_COLW = 128
_NPG = 8


def _make_sc_scatter(n, dim, num_clusters):
    info = plsc.get_sparse_core_info()
    nc, ns = info.num_cores, info.num_subcores
    khalf = num_clusters // 2
    per_pg = n // _NPG
    nchunks = per_pg // _CHUNK
    mesh = plsc.VectorSubcoreMesh(core_axis_name="c", subcore_axis_name="s")

    @functools.partial(
        pl.kernel, mesh=mesh,
        out_type=jax.ShapeDtypeStruct((_NPG, num_clusters, dim), jnp.float32),
        scratch_types=[
            pltpu.VMEM((khalf + 1, _COLW), jnp.float32),
            pltpu.VMEM((per_pg,), jnp.int32),
            pltpu.VMEM((_CHUNK, _COLW), jnp.float32),
        ],
    )
    def sc_scatter(x_hbm, a_hbm, zeros_hbm, sums_out, acc, idx_v, x_v):
        c = lax.axis_index("c")
        s = lax.axis_index("s")
        kh = s // 8
        pg = s % 8
        lo = kh * khalf
        pltpu.sync_copy(zeros_hbm, acc)
        pltpu.sync_copy(a_hbm.at[pl.ds(pg * per_pg, per_pg)], idx_v)

        @pl.loop(0, nchunks)
        def _chunk(k):
            pltpu.sync_copy(
                x_hbm.at[pl.ds(pg * per_pg + k * _CHUNK, _CHUNK),
                         pl.ds(c * _COLW, _COLW)],
                x_v)

            @plsc.parallel_loop(0, _CHUNK // 16, unroll=2)
            def _grp(g):
                avec = idx_v[pl.ds(k * _CHUNK + g * 16, 16)]
                local_u = plsc.bitcast(avec - lo, jnp.uint32)
                acvec = plsc.bitcast(
                    jnp.minimum(local_u, jnp.uint32(khalf)), jnp.int32)
                for l in range(16):
                    a = acvec[l]
                    for j in range(_COLW // 16):
                        plsc.addupdate(acc.at[a, pl.ds(j * 16, 16)],
                                       x_v[g * 16 + l, pl.ds(j * 16, 16)])

        pltpu.sync_copy(
            acc.at[pl.ds(0, khalf)],
            sums_out.at[pg, pl.ds(kh * khalf, khalf),
                        pl.ds(c * _COLW, _COLW)])

    return sc_scatter


def _combine_body(sums_ref, cnt_ref, c_ref, centers_out_ref):
    sums = jnp.sum(sums_ref[...], axis=0)
    counts = cnt_ref[:, 0:1]
    means = sums / jnp.maximum(counts, 1.0)
    centers_out_ref[...] = jnp.where(counts > 0.0, means, c_ref[...])


def _tc_combine(sums, cnt, centers):
    num_clusters, dim = centers.shape
    return pl.pallas_call(
        _combine_body,
        out_shape=jax.ShapeDtypeStruct((num_clusters, dim), jnp.float32),
    )(sums, cnt, centers)


@jax.jit
def kernel(x, cluster_centers):
    n, dim = x.shape
    num_clusters = cluster_centers.shape[0]

    assignments, counts8 = _tc_argmin(x, cluster_centers)
    zeros = jnp.zeros((num_clusters // 2 + 1, _COLW), jnp.float32)
    sc_scatter = _make_sc_scatter(n, dim, num_clusters)
    sums = sc_scatter(x, assignments, zeros)
    new_centers = _tc_combine(sums, counts8, cluster_centers)
    return new_centers, counts8[:, 0]
